# baseline (speedup 1.0000x reference)
import jax
import jax.numpy as jnp
from jax import lax
from jax.experimental import pallas as pl

_PATCH = 16
_K_TOP = 16


def _scores_body(p_ref, s_ref):
    p = p_ref[0]
    n, d = p.shape
    pb = p.astype(jnp.bfloat16)
    g = lax.dot_general(pb, pb, (((1,), (1,)), ((), ())),
                        preferred_element_type=jnp.float32)
    p2 = p * p
    sq_col = jnp.sum(p2, axis=1, keepdims=True)
    ones = jnp.ones((1, d), jnp.float32)
    sq_row = lax.dot_general(ones, p2, (((1,), (1,)), ((), ())),
                             precision=lax.Precision.HIGHEST)
    d2 = jnp.maximum(sq_col + sq_row - 2.0 * g, 0.0)
    s_ref[0] = jnp.sum(d2, axis=0, keepdims=True) * (1.0 / n)


def _topk_body(s_ref, out_ref):
    s = s_ref[...]
    B, n = s.shape
    lane = lax.broadcasted_iota(jnp.int32, (B, n), 1)
    lane_k = lax.broadcasted_iota(jnp.int32, (B, _K_TOP), 1)
    acc = jnp.zeros((B, _K_TOP), jnp.int32)
    for t in range(_K_TOP):
        m = jnp.max(s, axis=1, keepdims=True)
        idx = jnp.min(jnp.where(s == m, lane, n), axis=1, keepdims=True)
        acc = jnp.where(lane_k == t, idx, acc)
        s = jnp.where(lane == idx, -1.0, s)
    out_ref[...] = acc


def kernel(observed_data, observed_mask):
    del observed_mask
    B, K, L = observed_data.shape
    n = L // _PATCH
    d = K * _PATCH
    n_chunks = 4
    cb = B // n_chunks
    score_chunks = []
    for c in range(n_chunks):
        xc = lax.slice_in_dim(observed_data, c * cb, (c + 1) * cb, axis=0)
        patches = xc.swapaxes(1, 2).reshape(cb, n, d)
        sc = pl.pallas_call(
            _scores_body,
            grid=(cb,),
            in_specs=[pl.BlockSpec((1, n, d), lambda b: (b, 0, 0))],
            out_specs=pl.BlockSpec((1, 1, n), lambda b: (b, 0, 0)),
            out_shape=jax.ShapeDtypeStruct((cb, 1, n), jnp.float32),
        )(patches)
        score_chunks.append(sc)
    scores = jnp.concatenate(score_chunks, axis=0)
    out = pl.pallas_call(
        _topk_body,
        in_specs=[pl.BlockSpec((B, n), lambda: (0, 0))],
        out_specs=pl.BlockSpec((B, _K_TOP), lambda: (0, 0)),
        out_shape=jax.ShapeDtypeStruct((B, _K_TOP), jnp.int32),
    )(scores.reshape(B, n))
    return out


# speedup vs baseline: 1.3044x; 1.3044x over previous
import jax
import jax.numpy as jnp
from jax import lax
from jax.experimental import pallas as pl
from jax.experimental.pallas import tpu as pltpu

_PATCH = 16
_K_TOP = 16


def _body(p_ref, out_ref, s_scr):
    b = pl.program_id(0)
    nb = pl.num_programs(0)
    p = p_ref[0]
    n, d = p.shape
    pb = p.astype(jnp.bfloat16)
    g = lax.dot_general(pb, pb, (((1,), (1,)), ((), ())),
                        preferred_element_type=jnp.float32)
    p2 = p * p
    sq_col = jnp.sum(p2, axis=1, keepdims=True)
    ones = jnp.ones((1, d), jnp.float32)
    sq_row = lax.dot_general(ones, p2, (((1,), (1,)), ((), ())),
                             precision=lax.Precision.HIGHEST)
    d2 = jnp.maximum(sq_col + sq_row - 2.0 * g, 0.0)
    s_scr[pl.ds(b, 1), :] = jnp.sum(d2, axis=0, keepdims=True) * (1.0 / n)

    @pl.when(b == nb - 1)
    def _topk():
        s = s_scr[...]
        B = s.shape[0]
        lane = lax.broadcasted_iota(jnp.int32, (B, n), 1)
        lane_k = lax.broadcasted_iota(jnp.int32, (B, _K_TOP), 1)
        acc = jnp.zeros((B, _K_TOP), jnp.int32)
        for t in range(_K_TOP):
            m = jnp.max(s, axis=1, keepdims=True)
            idx = jnp.min(jnp.where(s == m, lane, n), axis=1, keepdims=True)
            acc = jnp.where(lane_k == t, idx, acc)
            s = jnp.where(lane == idx, -1.0, s)
        out_ref[...] = acc


def kernel(observed_data, observed_mask):
    del observed_mask
    B, K, L = observed_data.shape
    n = L // _PATCH
    d = K * _PATCH
    patches = observed_data.swapaxes(1, 2).reshape(B, n, d)
    out = pl.pallas_call(
        _body,
        grid=(B,),
        in_specs=[pl.BlockSpec((1, n, d), lambda b: (b, 0, 0))],
        out_specs=pl.BlockSpec((B, _K_TOP), lambda b: (0, 0)),
        out_shape=jax.ShapeDtypeStruct((B, _K_TOP), jnp.int32),
        scratch_shapes=[pltpu.VMEM((B, n), jnp.float32)],
    )(patches)
    return out


# speedup vs baseline: 1.3073x; 1.0022x over previous
import jax
import jax.numpy as jnp
from jax import lax
from jax.experimental import pallas as pl
from jax.experimental.pallas import tpu as pltpu

_PATCH = 16
_K_TOP = 16


def _body(t_ref, b_ref, out_ref, s_scr):
    b = pl.program_id(0)
    nb = pl.num_programs(0)
    p = jnp.concatenate([t_ref[0], b_ref[0]], axis=0)
    n, d = p.shape
    pb = p.astype(jnp.bfloat16)
    g = lax.dot_general(pb, pb, (((1,), (1,)), ((), ())),
                        preferred_element_type=jnp.float32)
    p2 = p * p
    sq_col = jnp.sum(p2, axis=1, keepdims=True)
    ones = jnp.ones((1, d), jnp.float32)
    sq_row = lax.dot_general(ones, p2, (((1,), (1,)), ((), ())),
                             precision=lax.Precision.HIGHEST)
    d2 = jnp.maximum(sq_col + sq_row - 2.0 * g, 0.0)
    s_scr[pl.ds(b, 1), :] = jnp.sum(d2, axis=0, keepdims=True) * (1.0 / n)

    @pl.when(b == nb - 1)
    def _topk():
        s = s_scr[...]
        B = s.shape[0]
        lane = lax.broadcasted_iota(jnp.int32, (B, n), 1)
        lane_k = lax.broadcasted_iota(jnp.int32, (B, _K_TOP), 1)
        acc = jnp.zeros((B, _K_TOP), jnp.int32)
        for t in range(_K_TOP):
            m = jnp.max(s, axis=1, keepdims=True)
            idx = jnp.min(jnp.where(s == m, lane, n), axis=1, keepdims=True)
            acc = jnp.where(lane_k == t, idx, acc)
            s = jnp.where(lane == idx, -1.0, s)
        out_ref[...] = acc


def kernel(observed_data, observed_mask):
    del observed_mask
    B, K, L = observed_data.shape
    n = L // _PATCH
    d = K * _PATCH
    patches = observed_data.swapaxes(1, 2).reshape(B, n, d)
    h = n // 2
    out = pl.pallas_call(
        _body,
        grid=(B,),
        in_specs=[pl.BlockSpec((1, h, d), lambda b: (b, 0, 0)),
                  pl.BlockSpec((1, h, d), lambda b: (b, 1, 0))],
        out_specs=pl.BlockSpec((B, _K_TOP), lambda b: (0, 0)),
        out_shape=jax.ShapeDtypeStruct((B, _K_TOP), jnp.int32),
        scratch_shapes=[pltpu.VMEM((B, n), jnp.float32)],
    )(patches, patches)
    return out


# speedup vs baseline: 1.4336x; 1.0966x over previous
import jax
import jax.numpy as jnp
from jax import lax
from jax.experimental import pallas as pl
from jax.experimental.pallas import tpu as pltpu

_PATCH = 16
_K_TOP = 16


def _body(t_ref, b_ref, out_ref, s_scr):
    b = pl.program_id(0)
    nb = pl.num_programs(0)
    p = jnp.concatenate([t_ref[0], b_ref[0]], axis=0)
    n, d = p.shape
    pb = p.astype(jnp.bfloat16)
    g = lax.dot_general(pb, pb, (((1,), (1,)), ((), ())),
                        preferred_element_type=jnp.float32)
    p2 = p * p
    sq_col = jnp.sum(p2, axis=1, keepdims=True)
    sq_row = jnp.transpose(sq_col)
    d2 = jnp.maximum(sq_col + sq_row - 2.0 * g, 0.0)
    s_scr[pl.ds(b, 1), :] = jnp.sum(d2, axis=0, keepdims=True) * (1.0 / n)

    @pl.when(b == nb - 1)
    def _topk():
        s = s_scr[...]
        B = s.shape[0]
        lane = lax.broadcasted_iota(jnp.int32, (B, n), 1)
        lane_k = lax.broadcasted_iota(jnp.int32, (B, _K_TOP), 1)
        acc = jnp.zeros((B, _K_TOP), jnp.int32)
        for t in range(_K_TOP):
            m = jnp.max(s, axis=1, keepdims=True)
            idx = jnp.min(jnp.where(s == m, lane, n), axis=1, keepdims=True)
            acc = jnp.where(lane_k == t, idx, acc)
            s = jnp.where(lane == idx, -1.0, s)
        out_ref[...] = acc


def kernel(observed_data, observed_mask):
    del observed_mask
    B, K, L = observed_data.shape
    n = L // _PATCH
    d = K * _PATCH
    patches = observed_data.swapaxes(1, 2).reshape(B, n, d)
    h = n // 2
    out = pl.pallas_call(
        _body,
        grid=(B,),
        in_specs=[pl.BlockSpec((1, h, d), lambda b: (b, 0, 0)),
                  pl.BlockSpec((1, h, d), lambda b: (b, 1, 0))],
        out_specs=pl.BlockSpec((B, _K_TOP), lambda b: (0, 0)),
        out_shape=jax.ShapeDtypeStruct((B, _K_TOP), jnp.int32),
        scratch_shapes=[pltpu.VMEM((B, n), jnp.float32)],
    )(patches, patches)
    return out
